# baseline (speedup 1.0000x reference)
import functools

import jax
import jax.numpy as jnp
from jax import lax
from jax.experimental import pallas as pl
from jax.experimental.pallas import tpu as pltpu
from jax.experimental.pallas import tpu_sc as plsc

N = 65536
L = 16
NC = 2
NS = 16
E = N // (NC * NS)
ITERS = E // L
U = 8

_mesh = plsc.VectorSubcoreMesh(
    core_axis_name="c", subcore_axis_name="s", num_cores=NC)


@functools.partial(
    pl.kernel,
    out_type=jax.ShapeDtypeStruct((NC, L), jnp.float32),
    mesh=_mesh,
    scratch_types=[
        pltpu.VMEM((E,), jnp.float32),
        pltpu.VMEM((E,), jnp.float32),
        pltpu.VMEM((E,), jnp.int32),
        pltpu.VMEM((2, L), jnp.float32),
        pltpu.VMEM((NS, 2, L), jnp.float32),
        pltpu.VMEM((L,), jnp.float32),
        pltpu.VMEM_SHARED((NS, 2, L), jnp.float32),
        pltpu.SemaphoreType.DMA,
        pltpu.SemaphoreType.DMA,
        pltpu.SemaphoreType.DMA,
    ],
    compiler_params=pltpu.CompilerParams(needs_layout_passes=False),
)
def _loss_kernel(a_hbm, b_hbm, lab_hbm, out_hbm, a_v, b_v, lab_v, part_v,
                 all_v, res_v, shared, sem_a, sem_b, sem_l):
    cid = lax.axis_index("c")
    sid = lax.axis_index("s")
    wid = sid * NC + cid
    cp_a = pltpu.async_copy(a_hbm.at[pl.ds(wid * E, E)], a_v, sem_a)
    cp_b = pltpu.async_copy(b_hbm.at[pl.ds(wid * E, E)], b_v, sem_b)
    cp_l = pltpu.async_copy(lab_hbm.at[pl.ds(wid * E, E)], lab_v, sem_l)
    cp_a.wait()
    cp_b.wait()
    cp_l.wait()

    def one(j, acc_l, acc_c):
        av = a_v[pl.ds(j * L, L)]
        bv = b_v[pl.ds(j * L, L)]
        lab = lab_v[pl.ds(j * L, L)]
        ev = bv - av
        t = jnp.where(lab == 1, -ev, ev)
        u = jnp.exp(-jnp.abs(t))
        z = u / (u + 2.0)
        z2 = z * z
        p = 2.0 + z2 * (2.0 / 3.0 + z2 * (2.0 / 5.0 + z2 * (2.0 / 7.0)))
        ce = jnp.maximum(t, 0.0) + z * p
        mf = jnp.where(lab != -1, 1.0, 0.0)
        return acc_l + ce * mf, acc_c + mf

    def body(trip, carry):
        accs = list(carry)
        j0 = trip * U
        for k in range(U):
            accs[2 * k], accs[2 * k + 1] = one(
                j0 + k, accs[2 * k], accs[2 * k + 1])
        return tuple(accs)

    zero = jnp.zeros((L,), jnp.float32)
    accs = lax.fori_loop(0, ITERS // U, body, (zero,) * (2 * U))
    acc_l = accs[0]
    acc_c = accs[1]
    for k in range(1, U):
        acc_l = acc_l + accs[2 * k]
        acc_c = acc_c + accs[2 * k + 1]

    part_v[0, :] = acc_l
    part_v[1, :] = acc_c
    pltpu.sync_copy(part_v, shared.at[sid])
    plsc.subcore_barrier()

    @pl.when(sid == 0)
    def _():
        pltpu.sync_copy(shared, all_v)
        tl = all_v[0, 0, :]
        tc = all_v[0, 1, :]
        for s in range(1, NS):
            tl = tl + all_v[s, 0, :]
            tc = tc + all_v[s, 1, :]
        s_l = jnp.full((L,), jnp.sum(tl), jnp.float32)
        s_c = jnp.full((L,), jnp.sum(tc), jnp.float32)
        lane = lax.iota(jnp.int32, L)
        res_v[...] = jnp.where(lane == 0, s_l, s_c)
        pltpu.sync_copy(res_v, out_hbm.at[cid])


def kernel(out_cls, labels):
    a = out_cls[:, 0]
    b = out_cls[:, 1]
    lab = labels.reshape(-1).astype(jnp.int32)
    o = _loss_kernel(a, b, lab)
    s_l = o[0, 0] + o[1, 0]
    s_c = o[0, 1] + o[1, 1]
    return s_l / jnp.maximum(s_c, 1.0)


# speedup vs baseline: 1.1651x; 1.1651x over previous
import functools

import numpy as np

import jax
import jax.numpy as jnp
from jax import lax
from jax.experimental import pallas as pl
from jax.experimental.pallas import tpu as pltpu
from jax.experimental.pallas import tpu_sc as plsc

N = 65536
L = 16
NW = 16
E = N // NW
ITERS = E // L
U = 8

TBL_STEP = 64.0
TBL_MAX_IDX = 1087.0
TBL_PAD = 1152

_G_TABLE = np.log1p(
    np.exp(-np.arange(TBL_PAD, dtype=np.float64) / TBL_STEP)
).astype(np.float32)

_mesh = plsc.VectorSubcoreMesh(
    core_axis_name="c", subcore_axis_name="s", num_cores=1)


@functools.partial(
    pl.kernel,
    out_type=(
        jax.ShapeDtypeStruct((L,), jnp.float32),
        jax.ShapeDtypeStruct((NW, 2, L), jnp.float32),
    ),
    mesh=_mesh,
    scratch_types=[
        pltpu.VMEM((E,), jnp.float32),
        pltpu.VMEM((E,), jnp.float32),
        pltpu.VMEM((E,), jnp.int32),
        pltpu.VMEM((TBL_PAD,), jnp.float32),
        pltpu.VMEM((2, L), jnp.float32),
        pltpu.VMEM((NW, 2, L), jnp.float32),
        pltpu.VMEM((L,), jnp.float32),
        pltpu.SemaphoreType.DMA,
        pltpu.SemaphoreType.DMA,
        pltpu.SemaphoreType.DMA,
        pltpu.SemaphoreType.DMA,
    ],
    compiler_params=pltpu.CompilerParams(needs_layout_passes=False),
)
def _loss_kernel(a_hbm, b_hbm, lab_hbm, tbl_hbm, out_hbm, parts_hbm, a_v,
                 b_v, lab_v, tbl_v, part_v, all_v, res_v, sem_a, sem_b,
                 sem_l, sem_t):
    sid = lax.axis_index("s")
    cp_t = pltpu.async_copy(tbl_hbm, tbl_v, sem_t)
    cp_a = pltpu.async_copy(a_hbm.at[pl.ds(sid * E, E)], a_v, sem_a)
    cp_b = pltpu.async_copy(b_hbm.at[pl.ds(sid * E, E)], b_v, sem_b)
    cp_l = pltpu.async_copy(lab_hbm.at[pl.ds(sid * E, E)], lab_v, sem_l)
    cp_t.wait()
    cp_a.wait()
    cp_b.wait()
    cp_l.wait()

    def one(j, acc_l, acc_c):
        av = a_v[pl.ds(j * L, L)]
        bv = b_v[pl.ds(j * L, L)]
        lab = lab_v[pl.ds(j * L, L)]
        ev = bv - av
        t = jnp.where(lab == 1, -ev, ev)
        s = jnp.minimum(jnp.abs(t) * TBL_STEP, TBL_MAX_IDX)
        i = s.astype(jnp.int32)
        f = s - i.astype(jnp.float32)
        g0 = plsc.load_gather(tbl_v, [i])
        g1 = plsc.load_gather(tbl_v, [i + 1])
        ce = jnp.maximum(t, 0.0) + g0 + f * (g1 - g0)
        mf = jnp.where(lab != -1, 1.0, 0.0)
        return acc_l + ce * mf, acc_c + mf

    def body(trip, carry):
        accs = list(carry)
        j0 = trip * U
        for k in range(U):
            accs[2 * k], accs[2 * k + 1] = one(
                j0 + k, accs[2 * k], accs[2 * k + 1])
        return tuple(accs)

    zero = jnp.zeros((L,), jnp.float32)
    accs = lax.fori_loop(0, ITERS // U, body, (zero,) * (2 * U))
    acc_l = accs[0]
    acc_c = accs[1]
    for k in range(1, U):
        acc_l = acc_l + accs[2 * k]
        acc_c = acc_c + accs[2 * k + 1]

    part_v[0, :] = acc_l
    part_v[1, :] = acc_c
    pltpu.sync_copy(part_v, parts_hbm.at[sid])
    plsc.subcore_barrier()

    @pl.when(sid == 0)
    def _():
        pltpu.sync_copy(parts_hbm, all_v)
        tl = all_v[0, 0, :]
        tc = all_v[0, 1, :]
        for s in range(1, NW):
            tl = tl + all_v[s, 0, :]
            tc = tc + all_v[s, 1, :]
        s_l = jnp.full((L,), jnp.sum(tl), jnp.float32)
        d = jnp.maximum(jnp.full((L,), jnp.sum(tc), jnp.float32), 1.0)
        r = 1.0 / d
        r = r * (2.0 - d * r)
        res_v[...] = s_l * r
        pltpu.sync_copy(res_v, out_hbm)


def kernel(out_cls, labels):
    a = out_cls[:, 0]
    b = out_cls[:, 1]
    lab = labels.reshape(-1).astype(jnp.int32)
    out, _ = _loss_kernel(a, b, lab, jnp.asarray(_G_TABLE))
    return out[0]
